# Initial kernel scaffold; baseline (speedup 1.0000x reference)
import jax
import jax.numpy as jnp
from jax.experimental import pallas as pl


def kernel(x, edge_index, edge_attr, W0, b0, W1, b1, W2, b2, W3, b3, Wf, bf):
    raise NotImplementedError("write your pallas kernel here")


# speedup vs baseline: 6.3347x; 6.3347x over previous
import functools

import jax
import jax.numpy as jnp
from jax import lax
from jax.experimental import pallas as pl
from jax.experimental.pallas import tpu as pltpu
from jax.experimental.pallas import tpu_sc as plsc

_N = 10000
_NP = 10240
_E = 320000
_EP = 327680
_EROWS = _EP // 128
_BN = 1024
_GRID = _NP // _BN


def _mesh():
    return plsc.VectorSubcoreMesh(core_axis_name="c", subcore_axis_name="s")


_SC_PARAMS = pltpu.CompilerParams(use_tc_tiling_on_sc=False)


def _make_msg_edge_split(F):
    rows_per_tile = _EROWS // 32
    n_super = rows_per_tile // 16
    kf = F // 16

    @functools.partial(
        pl.kernel,
        out_type=jax.ShapeDtypeStruct((2, _NP, F), jnp.float32),
        mesh=_mesh(),
        compiler_params=_SC_PARAMS,
        scratch_types=[
            pltpu.VMEM((16, 128), jnp.int32),
            pltpu.VMEM((16, 128), jnp.int32),
            pltpu.VMEM((16, 128), jnp.float32),
            pltpu.VMEM((128, F), jnp.float32),
            pltpu.VMEM_SHARED((_NP, F), jnp.float32),
            pltpu.SemaphoreType.DMA,
        ],
    )
    def k(g, src2d, dst2d, w2d, out, srcb, dstb, wb, rows, acc, sem):
        c = lax.axis_index("c")
        s = lax.axis_index("s")
        t = c * 16 + s

        def zr(e, _):
            for q in range(kf):
                rows[e, pl.ds(q * 16, 16)] = jnp.zeros((16,), jnp.float32)
            return 0
        lax.fori_loop(0, 128, zr, 0)
        for z in range(5):
            pltpu.sync_copy(rows, acc.at[pl.ds(s * 640 + z * 128, 128)])
        plsc.subcore_barrier()

        row0 = t * rows_per_tile

        def super_body(sup, _):
            r0 = row0 + sup * 16
            pltpu.sync_copy(src2d.at[pl.ds(r0, 16)], srcb)
            pltpu.sync_copy(dst2d.at[pl.ds(r0, 16)], dstb)
            pltpu.sync_copy(w2d.at[pl.ds(r0, 16)], wb)

            def sub_body(j, _):
                pltpu.async_copy(g.at[srcb.at[j]], rows, sem).wait()

                def mul(eb, _):
                    wv = wb[j, pl.ds(eb * 16, 16)]
                    for l in range(16):
                        ws = wv[l]
                        e = eb * 16 + l
                        for q in range(kf):
                            sl = pl.ds(q * 16, 16)
                            rows[e, sl] = rows[e, sl] * ws
                    return 0
                lax.fori_loop(0, 8, mul, 0)
                pltpu.sync_copy(rows, acc.at[dstb.at[j]], add=True)
                return 0
            lax.fori_loop(0, 16, sub_body, 0)
            return 0
        lax.fori_loop(0, n_super, super_body, 0)

        plsc.subcore_barrier()
        pltpu.sync_copy(acc.at[pl.ds(s * 640, 640)],
                        out.at[c, pl.ds(s * 640, 640)])
    return k


def _make_msg_col_split():
    Fc = 128
    kf = Fc // 16
    rows_per_tile = _EROWS // 16
    n_super = rows_per_tile // 16

    @functools.partial(
        pl.kernel,
        out_type=jax.ShapeDtypeStruct((2 * _NP, Fc), jnp.float32),
        mesh=_mesh(),
        compiler_params=_SC_PARAMS,
        scratch_types=[
            pltpu.VMEM((16, 128), jnp.int32),
            pltpu.VMEM((16, 128), jnp.int32),
            pltpu.VMEM((16, 128), jnp.float32),
            pltpu.VMEM((128, Fc), jnp.float32),
            pltpu.VMEM_SHARED((_NP, Fc), jnp.float32),
            pltpu.SemaphoreType.DMA,
        ],
    )
    def k(g, src2d, dst2d, w2d, out, srcb, dstb, wb, rows, acc, sem):
        c = lax.axis_index("c")
        s = lax.axis_index("s")
        goff = c * _NP

        def zr(e, _):
            for q in range(kf):
                rows[e, pl.ds(q * 16, 16)] = jnp.zeros((16,), jnp.float32)
            return 0
        lax.fori_loop(0, 128, zr, 0)
        for z in range(5):
            pltpu.sync_copy(rows, acc.at[pl.ds(s * 640 + z * 128, 128)])
        plsc.subcore_barrier()

        row0 = s * rows_per_tile

        def super_body(sup, _):
            r0 = row0 + sup * 16
            pltpu.sync_copy(src2d.at[pl.ds(r0, 16)], srcb)
            pltpu.sync_copy(dst2d.at[pl.ds(r0, 16)], dstb)
            pltpu.sync_copy(w2d.at[pl.ds(r0, 16)], wb)

            def off(r, _):
                for q in range(8):
                    sl = pl.ds(q * 16, 16)
                    srcb[r, sl] = srcb[r, sl] + goff
                return 0
            lax.fori_loop(0, 16, off, 0)

            def sub_body(j, _):
                pltpu.async_copy(g.at[srcb.at[j]], rows, sem).wait()

                def mul(eb, _):
                    wv = wb[j, pl.ds(eb * 16, 16)]
                    for l in range(16):
                        ws = wv[l]
                        e = eb * 16 + l
                        for q in range(kf):
                            sl = pl.ds(q * 16, 16)
                            rows[e, sl] = rows[e, sl] * ws
                    return 0
                lax.fori_loop(0, 8, mul, 0)
                pltpu.sync_copy(rows, acc.at[dstb.at[j]], add=True)
                return 0
            lax.fori_loop(0, 16, sub_body, 0)
            return 0
        lax.fori_loop(0, n_super, super_body, 0)

        plsc.subcore_barrier()
        pltpu.sync_copy(acc.at[pl.ds(s * 640, 640)],
                        out.at[pl.ds(goff + s * 640, 640)])
    return k


_msg16 = _make_msg_edge_split(16)
_msg64 = _make_msg_edge_split(64)
_msg128 = _make_msg_edge_split(128)
_msg256 = _make_msg_col_split()


def _row_spec(w, ndim2=True):
    return pl.BlockSpec((_BN, w), lambda i: (i, 0))


def _tc0(degp0, degp1, x, w0):
    def body(d0, d1, xr, wr, dinv_o, g0_o):
        deg = d0[:, 0:1] + d1[:, 0:1] + 1.0
        di = lax.rsqrt(deg)
        dinv_o[:, :] = di
        m = jnp.dot(xr[:, :], wr[:, :], preferred_element_type=jnp.float32)
        g0_o[:, :] = di * m
    return pl.pallas_call(
        body,
        grid=(_GRID,),
        in_specs=[
            pl.BlockSpec((_BN, 16), lambda i: (i, 0)),
            pl.BlockSpec((_BN, 16), lambda i: (i, 0)),
            pl.BlockSpec((_BN, 128), lambda i: (i, 0)),
            pl.BlockSpec((128, 64), lambda i: (0, 0)),
        ],
        out_specs=[
            pl.BlockSpec((_BN, 1), lambda i: (i, 0)),
            pl.BlockSpec((_BN, 64), lambda i: (i, 0)),
        ],
        out_shape=[
            jax.ShapeDtypeStruct((_NP, 1), jnp.float32),
            jax.ShapeDtypeStruct((_NP, 64), jnp.float32),
        ],
    )(degp0, degp1, x, w0)


def _tc1(dinv, a0, a1, g0, b0):
    def body(dr, a0r, a1r, gr, br, g1_o):
        di = dr[:, :]
        h1 = jnp.maximum(di * (a0r[:, :] + a1r[:, :] + gr[:, :]) + br[:, :],
                         0.0)
        g1_o[:, :] = di * h1
    return pl.pallas_call(
        body,
        grid=(_GRID,),
        in_specs=[
            pl.BlockSpec((_BN, 1), lambda i: (i, 0)),
            pl.BlockSpec((_BN, 64), lambda i: (i, 0)),
            pl.BlockSpec((_BN, 64), lambda i: (i, 0)),
            pl.BlockSpec((_BN, 64), lambda i: (i, 0)),
            pl.BlockSpec((1, 64), lambda i: (0, 0)),
        ],
        out_specs=pl.BlockSpec((_BN, 64), lambda i: (i, 0)),
        out_shape=jax.ShapeDtypeStruct((_NP, 64), jnp.float32),
    )(dinv, a0, a1, g0, b0)


def _tc2(dinv, a0, a1, g1, w1, b1):
    def body(dr, a0r, a1r, gr, wr, br, g2_o):
        di = dr[:, :]
        z = di * (a0r[:, :] + a1r[:, :] + gr[:, :])
        h2 = jnp.maximum(
            jnp.dot(z, wr[:, :], preferred_element_type=jnp.float32)
            + br[:, :], 0.0)
        g2_o[:, :] = di * h2
    return pl.pallas_call(
        body,
        grid=(_GRID,),
        in_specs=[
            pl.BlockSpec((_BN, 1), lambda i: (i, 0)),
            pl.BlockSpec((_BN, 64), lambda i: (i, 0)),
            pl.BlockSpec((_BN, 64), lambda i: (i, 0)),
            pl.BlockSpec((_BN, 64), lambda i: (i, 0)),
            pl.BlockSpec((64, 128), lambda i: (0, 0)),
            pl.BlockSpec((1, 128), lambda i: (0, 0)),
        ],
        out_specs=pl.BlockSpec((_BN, 128), lambda i: (i, 0)),
        out_shape=jax.ShapeDtypeStruct((_NP, 128), jnp.float32),
    )(dinv, a0, a1, g1, w1, b1)


def _tc3(dinv, a0, a1, g2, w2, b2):
    def body(dr, a0r, a1r, gr, wr, br, g3_o):
        di = dr[:, :]
        z = di * (a0r[:, :] + a1r[:, :] + gr[:, :])
        h3 = jnp.maximum(
            jnp.dot(z, wr[:, :], preferred_element_type=jnp.float32)
            + br[:, :], 0.0)
        g3_o[:, :] = di * h3
    return pl.pallas_call(
        body,
        grid=(2, _GRID),
        in_specs=[
            pl.BlockSpec((_BN, 1), lambda c, i: (i, 0)),
            pl.BlockSpec((_BN, 128), lambda c, i: (i, 0)),
            pl.BlockSpec((_BN, 128), lambda c, i: (i, 0)),
            pl.BlockSpec((_BN, 128), lambda c, i: (i, 0)),
            pl.BlockSpec((128, 128), lambda c, i: (0, c)),
            pl.BlockSpec((1, 128), lambda c, i: (0, c)),
        ],
        out_specs=pl.BlockSpec((_BN, 128), lambda c, i: (c * _GRID + i, 0)),
        out_shape=jax.ShapeDtypeStruct((2 * _NP, 128), jnp.float32),
    )(dinv, a0, a1, g2, w2, b2)


def _tc4(dinv, acc3a, acc3b, g3a, g3b, w3, b3, wf, bf):
    def body(dr, aar, abr, gar, gbr, wr, br, wfr, bfr, o):
        di = dr[:, :]
        za = di * (aar[:, :] + gar[:, :])
        zb = di * (abr[:, :] + gbr[:, :])
        z = jnp.concatenate([za, zb], axis=1)
        h4 = jnp.maximum(
            jnp.dot(z, wr[:, :], preferred_element_type=jnp.float32)
            + br[:, :], 0.0)
        o[:, :] = jax.nn.sigmoid(
            jnp.dot(h4, wfr[:, :], preferred_element_type=jnp.float32)
            + bfr[:, :])
    return pl.pallas_call(
        body,
        grid=(_GRID,),
        in_specs=[
            pl.BlockSpec((_BN, 1), lambda i: (i, 0)),
            pl.BlockSpec((_BN, 128), lambda i: (i, 0)),
            pl.BlockSpec((_BN, 128), lambda i: (_GRID + i, 0)),
            pl.BlockSpec((_BN, 128), lambda i: (i, 0)),
            pl.BlockSpec((_BN, 128), lambda i: (_GRID + i, 0)),
            pl.BlockSpec((256, 256), lambda i: (0, 0)),
            pl.BlockSpec((1, 256), lambda i: (0, 0)),
            pl.BlockSpec((256, 1), lambda i: (0, 0)),
            pl.BlockSpec((1, 1), lambda i: (0, 0)),
        ],
        out_specs=pl.BlockSpec((_BN, 1), lambda i: (i, 0)),
        out_shape=jax.ShapeDtypeStruct((_NP, 1), jnp.float32),
    )(dinv, acc3a, acc3b, g3a, g3b, w3, b3, wf, bf)


def kernel(x, edge_index, edge_attr, W0, b0, W1, b1, W2, b2, W3, b3, Wf, bf):
    src = edge_index[0]
    dst = edge_index[1]
    padE = _EP - _E
    src2d = jnp.concatenate(
        [src, jnp.zeros((padE,), jnp.int32)]).reshape(_EROWS, 128)
    dst2d = jnp.concatenate(
        [dst, jnp.zeros((padE,), jnp.int32)]).reshape(_EROWS, 128)
    w2d = jnp.concatenate(
        [edge_attr, jnp.zeros((padE,), jnp.float32)]).reshape(_EROWS, 128)
    xp = jnp.pad(x, ((0, _NP - _N), (0, 0)))
    ones16 = jnp.ones((_NP, 16), jnp.float32)

    degp = _msg16(ones16, src2d, dst2d, w2d)
    dinv, g0 = _tc0(degp[0], degp[1], xp, W0)
    acc0 = _msg64(g0, src2d, dst2d, w2d)
    g1 = _tc1(dinv, acc0[0], acc0[1], g0, b0.reshape(1, -1))
    acc1 = _msg64(g1, src2d, dst2d, w2d)
    g2 = _tc2(dinv, acc1[0], acc1[1], g1, W1, b1.reshape(1, -1))
    acc2 = _msg128(g2, src2d, dst2d, w2d)
    g3c = _tc3(dinv, acc2[0], acc2[1], g2, W2, b2.reshape(1, -1))
    acc3c = _msg256(g3c, src2d, dst2d, w2d)
    out = _tc4(dinv, acc3c, acc3c, g3c, g3c, W3, b3.reshape(1, -1),
               Wf, bf.reshape(1, -1))
    return out[:_N]
